# baseline (speedup 1.0000x reference)
import functools

import jax
import jax.numpy as jnp
from jax import lax
from jax.experimental import pallas as pl
from jax.experimental.pallas import tpu as pltpu
from jax.experimental.pallas import tpu_sc as plsc

L = 16
ROWS = 16384
COLS = 512
NV = COLS // L
NC = 2
NS = 16
NW = NC * NS
RPW = ROWS // NW
CHUNK = 64
NCH = RPW // CHUNK


def _sort16(x, up):
    if up:
        return lax.sort(x, dimension=0)
    return plsc.sort_key_val(x, x, descending=True)[0]


def _bitonic_sort_vregs(xs, asc):
    nv = len(xs)

    def up_of(r, k):
        u = ((r * L) & k) == 0
        return u if asc else (not u)

    xs = [_sort16(x, up_of(r, L)) for r, x in enumerate(xs)]
    k = 2 * L
    while k <= nv * L:
        j = k // 2
        while j >= L:
            dv = j // L
            new = list(xs)
            for r in range(nv):
                r2 = r ^ dv
                if r2 > r:
                    up = up_of(r, k)
                    lo = jnp.minimum(xs[r], xs[r2])
                    hi = jnp.maximum(xs[r], xs[r2])
                    new[r], new[r2] = (lo, hi) if up else (hi, lo)
            xs = new
            j //= 2
        xs = [_sort16(x, up_of(r, k)) for r, x in enumerate(xs)]
        k *= 2
    return xs


def _merge_half(xs):
    nv = len(xs)
    j = nv * L // 2
    while j >= L:
        dv = j // L
        new = list(xs)
        for r in range(nv):
            r2 = r ^ dv
            if r2 > r:
                new[r] = jnp.minimum(xs[r], xs[r2])
                new[r2] = jnp.maximum(xs[r], xs[r2])
        xs = new
        j //= 2
    return [lax.sort(x, dimension=0) for x in xs]


def _losn_sc_body(
    data_hbm, wrev_hbm, bias_hbm, out_hbm, rows_v, w_v, bias_v, out_v, out_sc, dsem
):
    wid = lax.axis_index("s") * NC + lax.axis_index("c")
    base = wid * RPW
    pltpu.sync_copy(wrev_hbm, w_v)
    pltpu.sync_copy(bias_hbm, bias_v)
    bvec = bias_v[...]
    lanes = lax.broadcasted_iota(jnp.int32, (L,), 0)

    def _start(ch, slot):
        pltpu.async_copy(
            data_hbm.at[pl.ds((base + ch * CHUNK) * COLS, CHUNK * COLS)],
            rows_v.at[slot],
            dsem.at[slot],
        )

    _start(0, 0)

    def chunk_body(ch, carry):
        slot = ch % 2

        @pl.when(ch + 1 < NCH)
        def _():
            _start(ch + 1, (ch + 1) % 2)

        pltpu.make_async_copy(
            data_hbm.at[pl.ds(base * COLS, CHUNK * COLS)],
            rows_v.at[slot],
            dsem.at[slot],
        ).wait()

        def one_row(t):
            off = t * COLS
            nh = NV // 2
            h0 = [rows_v[slot, pl.ds(off + v * L, L)] for v in range(nh)]
            h0 = _bitonic_sort_vregs(h0, True)
            for v in range(nh):
                rows_v[slot, pl.ds(off + v * L, L)] = h0[v]
            h1 = [rows_v[slot, pl.ds(off + (nh + v) * L, L)] for v in range(nh)]
            h1 = _bitonic_sort_vregs(h1, False)
            his = []
            for v in range(nh):
                a = rows_v[slot, pl.ds(off + v * L, L)]
                his.append(jnp.maximum(a, h1[v]))
                rows_v[slot, pl.ds(off + v * L, L)] = jnp.minimum(a, h1[v])
            his = _merge_half(his)
            acc = his[0] * w_v[pl.ds(nh * L, L)]
            for v in range(1, nh):
                acc = acc + his[v] * w_v[pl.ds((nh + v) * L, L)]
            los = [rows_v[slot, pl.ds(off + v * L, L)] for v in range(nh)]
            los = _merge_half(los)
            for v in range(nh):
                acc = acc + los[v] * w_v[pl.ds(v * L, L)]
            return jnp.sum(acc)

        @plsc.parallel_loop(0, CHUNK, unroll=2)
        def _(t):
            total = one_row(t)
            out_sc[pl.ds((ch * CHUNK + t) * L, L)] = jnp.full((L,), total)

        return carry

    lax.fori_loop(0, NCH, chunk_body, jnp.zeros((), jnp.int32))

    def comp_body(g, carry):
        idx = g * (L * L) + lanes * L
        vals = plsc.load_gather(out_sc, [idx])
        out_v[pl.ds(g * L, L)] = vals + bvec
        return carry

    lax.fori_loop(0, RPW // L, comp_body, jnp.zeros((), jnp.int32))
    pltpu.sync_copy(out_v, out_hbm.at[pl.ds(base, RPW)])


@functools.lru_cache(maxsize=1)
def _build():
    mesh = plsc.VectorSubcoreMesh(
        core_axis_name="c", subcore_axis_name="s", num_cores=NC, num_subcores=NS
    )
    return pl.kernel(
        _losn_sc_body,
        out_type=jax.ShapeDtypeStruct((ROWS,), jnp.float32),
        mesh=mesh,
        compiler_params=pltpu.CompilerParams(needs_layout_passes=False),
        scratch_types=[
            pltpu.VMEM((2, CHUNK * COLS), jnp.float32),
            pltpu.VMEM((COLS,), jnp.float32),
            pltpu.VMEM((L,), jnp.float32),
            pltpu.VMEM((RPW,), jnp.float32),
            pltpu.VMEM((RPW * L,), jnp.float32),
            pltpu.SemaphoreType.DMA((2,)),
        ],
    )


def kernel(data, weights, bias):
    wrev = jnp.flip(weights.reshape(COLS))
    bias16 = jnp.broadcast_to(bias.reshape(()), (L,))
    out = _build()(data.reshape(ROWS * COLS), wrev, bias16)
    return out.reshape(ROWS, 1)


# speedup vs baseline: 1.0311x; 1.0311x over previous
import functools

import jax
import jax.numpy as jnp
from jax import lax
from jax.experimental import pallas as pl
from jax.experimental.pallas import tpu as pltpu
from jax.experimental.pallas import tpu_sc as plsc

L = 16
ROWS = 16384
COLS = 512
NV = COLS // L
NC = 2
NS = 16
NW = NC * NS
RPW = ROWS // NW
CHUNK = 64
NCH = RPW // CHUNK


def _sort16(x, up):
    if up:
        return lax.sort(x, dimension=0)
    return -lax.sort(-x, dimension=0)


def _bitonic_sort_vregs(xs, asc):
    nv = len(xs)

    def up_of(r, k):
        u = ((r * L) & k) == 0
        return u if asc else (not u)

    xs = [_sort16(x, up_of(r, L)) for r, x in enumerate(xs)]
    k = 2 * L
    while k <= nv * L:
        j = k // 2
        while j >= L:
            dv = j // L
            new = list(xs)
            for r in range(nv):
                r2 = r ^ dv
                if r2 > r:
                    up = up_of(r, k)
                    lo = jnp.minimum(xs[r], xs[r2])
                    hi = jnp.maximum(xs[r], xs[r2])
                    new[r], new[r2] = (lo, hi) if up else (hi, lo)
            xs = new
            j //= 2
        xs = [_sort16(x, up_of(r, k)) for r, x in enumerate(xs)]
        k *= 2
    return xs


def _merge_half(xs):
    nv = len(xs)
    j = nv * L // 2
    while j >= L:
        dv = j // L
        new = list(xs)
        for r in range(nv):
            r2 = r ^ dv
            if r2 > r:
                new[r] = jnp.minimum(xs[r], xs[r2])
                new[r2] = jnp.maximum(xs[r], xs[r2])
        xs = new
        j //= 2
    return [lax.sort(x, dimension=0) for x in xs]


def _losn_sc_body(
    data_hbm, wrev_hbm, bias_hbm, out_hbm, rows_v, w_v, bias_v, out_v, out_sc, dsem
):
    wid = lax.axis_index("s") * NC + lax.axis_index("c")
    base = wid * RPW
    pltpu.sync_copy(wrev_hbm, w_v)
    pltpu.sync_copy(bias_hbm, bias_v)
    bvec = bias_v[...]
    lanes = lax.broadcasted_iota(jnp.int32, (L,), 0)

    def _start(ch, slot):
        pltpu.async_copy(
            data_hbm.at[pl.ds((base + ch * CHUNK) * COLS, CHUNK * COLS)],
            rows_v.at[slot],
            dsem.at[slot],
        )

    _start(0, 0)

    def chunk_body(ch, carry):
        slot = ch % 2

        @pl.when(ch + 1 < NCH)
        def _():
            _start(ch + 1, (ch + 1) % 2)

        pltpu.make_async_copy(
            data_hbm.at[pl.ds(base * COLS, CHUNK * COLS)],
            rows_v.at[slot],
            dsem.at[slot],
        ).wait()

        def one_row(t):
            xs = [rows_v[slot, pl.ds(t * COLS + v * L, L)] for v in range(NV)]
            xs = _bitonic_sort_vregs(xs, True)
            acc = xs[0] * w_v[pl.ds(0, L)]
            for v in range(1, NV):
                acc = acc + xs[v] * w_v[pl.ds(v * L, L)]
            return jnp.sum(acc)

        @plsc.parallel_loop(0, CHUNK, unroll=1)
        def _(t):
            total = one_row(t)
            out_sc[pl.ds((ch * CHUNK + t) * L, L)] = jnp.full((L,), total)

        return carry

    lax.fori_loop(0, NCH, chunk_body, jnp.zeros((), jnp.int32))

    def comp_body(g, carry):
        idx = g * (L * L) + lanes * L
        vals = plsc.load_gather(out_sc, [idx])
        out_v[pl.ds(g * L, L)] = vals + bvec
        return carry

    lax.fori_loop(0, RPW // L, comp_body, jnp.zeros((), jnp.int32))
    pltpu.sync_copy(out_v, out_hbm.at[pl.ds(base, RPW)])


@functools.lru_cache(maxsize=1)
def _build():
    mesh = plsc.VectorSubcoreMesh(
        core_axis_name="c", subcore_axis_name="s", num_cores=NC, num_subcores=NS
    )
    return pl.kernel(
        _losn_sc_body,
        out_type=jax.ShapeDtypeStruct((ROWS,), jnp.float32),
        mesh=mesh,
        compiler_params=pltpu.CompilerParams(needs_layout_passes=False),
        scratch_types=[
            pltpu.VMEM((2, CHUNK * COLS), jnp.float32),
            pltpu.VMEM((COLS,), jnp.float32),
            pltpu.VMEM((L,), jnp.float32),
            pltpu.VMEM((RPW,), jnp.float32),
            pltpu.VMEM((RPW * L,), jnp.float32),
            pltpu.SemaphoreType.DMA((2,)),
        ],
    )


def kernel(data, weights, bias):
    wrev = jnp.flip(weights.reshape(COLS))
    bias16 = jnp.broadcast_to(bias.reshape(()), (L,))
    out = _build()(data.reshape(ROWS * COLS), wrev, bias16)
    return out.reshape(ROWS, 1)


# speedup vs baseline: 1.1359x; 1.1016x over previous
import functools

import jax
import jax.numpy as jnp
from jax import lax
from jax.experimental import pallas as pl
from jax.experimental.pallas import tpu as pltpu
from jax.experimental.pallas import tpu_sc as plsc

L = 16
ROWS = 16384
COLS = 512
NV = COLS // L
NC = 2
NS = 16
NW = NC * NS
RPW = ROWS // NW
CHUNK = 64
NCH = RPW // CHUNK


def _sort16(x, up):
    if up:
        return lax.sort(x, dimension=0)
    return plsc.sort_key_val(x, x, descending=True)[0]


def _bitonic_sort_vregs(xs, asc):
    nv = len(xs)

    def up_of(r, k):
        u = ((r * L) & k) == 0
        return u if asc else (not u)

    xs = [_sort16(x, up_of(r, L)) for r, x in enumerate(xs)]
    k = 2 * L
    while k <= nv * L:
        j = k // 2
        while j >= L:
            dv = j // L
            new = list(xs)
            for r in range(nv):
                r2 = r ^ dv
                if r2 > r:
                    up = up_of(r, k)
                    lo = jnp.minimum(xs[r], xs[r2])
                    hi = jnp.maximum(xs[r], xs[r2])
                    new[r], new[r2] = (lo, hi) if up else (hi, lo)
            xs = new
            j //= 2
        xs = [_sort16(x, up_of(r, k)) for r, x in enumerate(xs)]
        k *= 2
    return xs


def _merge_half(xs):
    nv = len(xs)
    j = nv * L // 2
    while j >= L:
        dv = j // L
        new = list(xs)
        for r in range(nv):
            r2 = r ^ dv
            if r2 > r:
                new[r] = jnp.minimum(xs[r], xs[r2])
                new[r2] = jnp.maximum(xs[r], xs[r2])
        xs = new
        j //= 2
    return [lax.sort(x, dimension=0) for x in xs]


def _losn_sc_body(
    data_hbm, wrev_hbm, bias_hbm, out_hbm, rows_v, w_v, bias_v, out_v, out_sc, dsem
):
    wid = lax.axis_index("s") * NC + lax.axis_index("c")
    base = wid * RPW
    pltpu.sync_copy(wrev_hbm, w_v)
    pltpu.sync_copy(bias_hbm, bias_v)
    bvec = bias_v[...]
    lanes = lax.broadcasted_iota(jnp.int32, (L,), 0)

    def _start(ch, slot):
        pltpu.async_copy(
            data_hbm.at[pl.ds((base + ch * CHUNK) * COLS, CHUNK * COLS)],
            rows_v.at[slot],
            dsem.at[slot],
        )

    _start(0, 0)

    def chunk_body(ch, carry):
        slot = ch % 2

        @pl.when(ch + 1 < NCH)
        def _():
            _start(ch + 1, (ch + 1) % 2)

        pltpu.make_async_copy(
            data_hbm.at[pl.ds(base * COLS, CHUNK * COLS)],
            rows_v.at[slot],
            dsem.at[slot],
        ).wait()

        def one_row(t):
            xs = [rows_v[slot, pl.ds(t * COLS + v * L, L)] for v in range(NV)]
            xs = _bitonic_sort_vregs(xs, True)
            acc = xs[0] * w_v[pl.ds(0, L)]
            for v in range(1, NV):
                acc = acc + xs[v] * w_v[pl.ds(v * L, L)]
            return jnp.sum(acc)

        @plsc.parallel_loop(0, CHUNK, unroll=1)
        def _(t):
            total = one_row(t)
            out_sc[pl.ds((ch * CHUNK + t) * L, L)] = jnp.full((L,), total)

        return carry

    lax.fori_loop(0, NCH, chunk_body, jnp.zeros((), jnp.int32))

    def comp_body(g, carry):
        idx = g * (L * L) + lanes * L
        vals = plsc.load_gather(out_sc, [idx])
        out_v[pl.ds(g * L, L)] = vals + bvec
        return carry

    lax.fori_loop(0, RPW // L, comp_body, jnp.zeros((), jnp.int32))
    pltpu.sync_copy(out_v, out_hbm.at[pl.ds(base, RPW)])


@functools.lru_cache(maxsize=1)
def _build():
    mesh = plsc.VectorSubcoreMesh(
        core_axis_name="c", subcore_axis_name="s", num_cores=NC, num_subcores=NS
    )
    return pl.kernel(
        _losn_sc_body,
        out_type=jax.ShapeDtypeStruct((ROWS,), jnp.float32),
        mesh=mesh,
        compiler_params=pltpu.CompilerParams(needs_layout_passes=False),
        scratch_types=[
            pltpu.VMEM((2, CHUNK * COLS), jnp.float32),
            pltpu.VMEM((COLS,), jnp.float32),
            pltpu.VMEM((L,), jnp.float32),
            pltpu.VMEM((RPW,), jnp.float32),
            pltpu.VMEM((RPW * L,), jnp.float32),
            pltpu.SemaphoreType.DMA((2,)),
        ],
    )


def kernel(data, weights, bias):
    wrev = jnp.flip(weights.reshape(COLS))
    bias16 = jnp.broadcast_to(bias.reshape(()), (L,))
    out = _build()(data.reshape(ROWS * COLS), wrev, bias16)
    return out.reshape(ROWS, 1)


# speedup vs baseline: 1.4925x; 1.3139x over previous
import functools

import jax
import jax.numpy as jnp
from jax import lax
from jax.experimental import pallas as pl
from jax.experimental.pallas import tpu as pltpu
from jax.experimental.pallas import tpu_sc as plsc

L = 16
ROWS = 16384
COLS = 512
NV = COLS // L
NC = 2
NS = 16
NW = NC * NS
RPW = ROWS // NW
CHUNK = 64
NCH = RPW // CHUNK


def _sort16(x, up):
    if up:
        return lax.sort(x, dimension=0)
    return plsc.sort_key_val(x, x, descending=True)[0]


def _bitonic_sort_vregs(xs, asc):
    nv = len(xs)

    def up_of(r, k):
        u = ((r * L) & k) == 0
        return u if asc else (not u)

    xs = [_sort16(x, up_of(r, L)) for r, x in enumerate(xs)]
    k = 2 * L
    while k <= nv * L:
        j = k // 2
        while j >= L:
            dv = j // L
            new = list(xs)
            for r in range(nv):
                r2 = r ^ dv
                if r2 > r:
                    up = up_of(r, k)
                    lo = jnp.minimum(xs[r], xs[r2])
                    hi = jnp.maximum(xs[r], xs[r2])
                    new[r], new[r2] = (lo, hi) if up else (hi, lo)
            xs = new
            j //= 2
        xs = [_sort16(x, up_of(r, k)) for r, x in enumerate(xs)]
        k *= 2
    return xs


def _merge_half(xs):
    nv = len(xs)
    j = nv * L // 2
    while j >= L:
        dv = j // L
        new = list(xs)
        for r in range(nv):
            r2 = r ^ dv
            if r2 > r:
                new[r] = jnp.minimum(xs[r], xs[r2])
                new[r2] = jnp.maximum(xs[r], xs[r2])
        xs = new
        j //= 2
    return [lax.sort(x, dimension=0) for x in xs]


def _losn_sc_body(
    data_hbm, wrev_hbm, bias_hbm, out_hbm, rows_v, w_v, bias_v, out_v, out_sc, dsem
):
    wid = lax.axis_index("s") * NC + lax.axis_index("c")
    base = wid * RPW
    pltpu.sync_copy(wrev_hbm, w_v)
    pltpu.sync_copy(bias_hbm, bias_v)
    bvec = bias_v[...]
    lanes = lax.broadcasted_iota(jnp.int32, (L,), 0)

    def _start(ch, slot):
        pltpu.async_copy(
            data_hbm.at[pl.ds(base + ch * CHUNK, CHUNK), :],
            rows_v.at[slot],
            dsem.at[slot],
        )

    _start(0, 0)

    def chunk_body(ch, carry):
        slot = ch % 2

        @pl.when(ch + 1 < NCH)
        def _():
            _start(ch + 1, (ch + 1) % 2)

        pltpu.make_async_copy(
            data_hbm.at[pl.ds(base, CHUNK), :],
            rows_v.at[slot],
            dsem.at[slot],
        ).wait()

        def one_row(t):
            xs = [rows_v[slot, t, pl.ds(v * L, L)] for v in range(NV)]
            xs = _bitonic_sort_vregs(xs, True)
            acc = xs[0] * w_v[pl.ds(0, L)]
            for v in range(1, NV):
                acc = acc + xs[v] * w_v[pl.ds(v * L, L)]
            return jnp.sum(acc)

        @plsc.parallel_loop(0, CHUNK, unroll=1)
        def _(t):
            total = one_row(t)
            out_sc[pl.ds((ch * CHUNK + t) * L, L)] = jnp.full((L,), total)

        return carry

    lax.fori_loop(0, NCH, chunk_body, jnp.zeros((), jnp.int32))

    def comp_body(g, carry):
        idx = g * (L * L) + lanes * L
        vals = plsc.load_gather(out_sc, [idx])
        out_v[pl.ds(g * L, L)] = vals + bvec
        return carry

    lax.fori_loop(0, RPW // L, comp_body, jnp.zeros((), jnp.int32))
    pltpu.sync_copy(out_v, out_hbm.at[pl.ds(base, RPW)])


@functools.lru_cache(maxsize=1)
def _build():
    mesh = plsc.VectorSubcoreMesh(
        core_axis_name="c", subcore_axis_name="s", num_cores=NC, num_subcores=NS
    )
    return pl.kernel(
        _losn_sc_body,
        out_type=jax.ShapeDtypeStruct((ROWS,), jnp.float32),
        mesh=mesh,
        compiler_params=pltpu.CompilerParams(needs_layout_passes=False),
        scratch_types=[
            pltpu.VMEM((2, CHUNK, COLS), jnp.float32),
            pltpu.VMEM((COLS,), jnp.float32),
            pltpu.VMEM((L,), jnp.float32),
            pltpu.VMEM((RPW,), jnp.float32),
            pltpu.VMEM((RPW * L,), jnp.float32),
            pltpu.SemaphoreType.DMA((2,)),
        ],
    )


def kernel(data, weights, bias):
    wrev = jnp.flip(weights.reshape(COLS))
    bias16 = jnp.broadcast_to(bias.reshape(()), (L,))
    out = _build()(data, wrev, bias16)
    return out.reshape(ROWS, 1)


# speedup vs baseline: 1.5132x; 1.0139x over previous
import functools

import jax
import jax.numpy as jnp
from jax import lax
from jax.experimental import pallas as pl
from jax.experimental.pallas import tpu as pltpu
from jax.experimental.pallas import tpu_sc as plsc

L = 16
ROWS = 16384
COLS = 512
NV = COLS // L
NC = 2
NS = 16
NW = NC * NS
RPW = ROWS // NW
CHUNK = 32
NCH = RPW // CHUNK


def _sort16(x, up):
    if up:
        return lax.sort(x, dimension=0)
    return plsc.sort_key_val(x, x, descending=True)[0]


def _bitonic_sort_vregs(xs, asc):
    nv = len(xs)

    def up_of(r, k):
        u = ((r * L) & k) == 0
        return u if asc else (not u)

    xs = [_sort16(x, up_of(r, L)) for r, x in enumerate(xs)]
    k = 2 * L
    while k <= nv * L:
        j = k // 2
        while j >= L:
            dv = j // L
            new = list(xs)
            for r in range(nv):
                r2 = r ^ dv
                if r2 > r:
                    up = up_of(r, k)
                    lo = jnp.minimum(xs[r], xs[r2])
                    hi = jnp.maximum(xs[r], xs[r2])
                    new[r], new[r2] = (lo, hi) if up else (hi, lo)
            xs = new
            j //= 2
        xs = [_sort16(x, up_of(r, k)) for r, x in enumerate(xs)]
        k *= 2
    return xs


def _merge_half(xs):
    nv = len(xs)
    j = nv * L // 2
    while j >= L:
        dv = j // L
        new = list(xs)
        for r in range(nv):
            r2 = r ^ dv
            if r2 > r:
                new[r] = jnp.minimum(xs[r], xs[r2])
                new[r2] = jnp.maximum(xs[r], xs[r2])
        xs = new
        j //= 2
    return [lax.sort(x, dimension=0) for x in xs]


def _losn_sc_body(
    data_hbm, wrev_hbm, bias_hbm, out_hbm, rows_v, w_v, bias_v, out_v, out_sc, dsem
):
    wid = lax.axis_index("s") * NC + lax.axis_index("c")
    base = wid * RPW
    pltpu.sync_copy(wrev_hbm, w_v)
    pltpu.sync_copy(bias_hbm, bias_v)
    bvec = bias_v[...]
    lanes = lax.broadcasted_iota(jnp.int32, (L,), 0)

    def _start(ch, slot):
        pltpu.async_copy(
            data_hbm.at[pl.ds(base + ch * CHUNK, CHUNK), :],
            rows_v.at[slot],
            dsem.at[slot],
        )

    _start(0, 0)

    def chunk_body(ch, carry):
        slot = ch % 2

        @pl.when(ch + 1 < NCH)
        def _():
            _start(ch + 1, (ch + 1) % 2)

        pltpu.make_async_copy(
            data_hbm.at[pl.ds(base, CHUNK), :],
            rows_v.at[slot],
            dsem.at[slot],
        ).wait()

        def one_row(t):
            xs = [rows_v[slot, t, pl.ds(v * L, L)] for v in range(NV)]
            xs = _bitonic_sort_vregs(xs, True)
            acc = xs[0] * w_v[pl.ds(0, L)]
            for v in range(1, NV):
                acc = acc + xs[v] * w_v[pl.ds(v * L, L)]
            return jnp.sum(acc)

        @plsc.parallel_loop(0, CHUNK, unroll=1)
        def _(t):
            total = one_row(t)
            out_sc[pl.ds((ch * CHUNK + t) * L, L)] = jnp.full((L,), total)

        return carry

    lax.fori_loop(0, NCH, chunk_body, jnp.zeros((), jnp.int32))

    def comp_body(g, carry):
        idx = g * (L * L) + lanes * L
        vals = plsc.load_gather(out_sc, [idx])
        out_v[pl.ds(g * L, L)] = vals + bvec
        return carry

    lax.fori_loop(0, RPW // L, comp_body, jnp.zeros((), jnp.int32))
    pltpu.sync_copy(out_v, out_hbm.at[pl.ds(base, RPW)])


@functools.lru_cache(maxsize=1)
def _build():
    mesh = plsc.VectorSubcoreMesh(
        core_axis_name="c", subcore_axis_name="s", num_cores=NC, num_subcores=NS
    )
    return pl.kernel(
        _losn_sc_body,
        out_type=jax.ShapeDtypeStruct((ROWS,), jnp.float32),
        mesh=mesh,
        compiler_params=pltpu.CompilerParams(needs_layout_passes=False),
        scratch_types=[
            pltpu.VMEM((2, CHUNK, COLS), jnp.float32),
            pltpu.VMEM((COLS,), jnp.float32),
            pltpu.VMEM((L,), jnp.float32),
            pltpu.VMEM((RPW,), jnp.float32),
            pltpu.VMEM((RPW * L,), jnp.float32),
            pltpu.SemaphoreType.DMA((2,)),
        ],
    )


def kernel(data, weights, bias):
    wrev = jnp.flip(weights.reshape(COLS))
    bias16 = jnp.broadcast_to(bias.reshape(()), (L,))
    out = _build()(data, wrev, bias16)
    return out.reshape(ROWS, 1)


# speedup vs baseline: 1.5241x; 1.0072x over previous
import functools

import jax
import jax.numpy as jnp
from jax import lax
from jax.experimental import pallas as pl
from jax.experimental.pallas import tpu as pltpu
from jax.experimental.pallas import tpu_sc as plsc

L = 16
ROWS = 16384
COLS = 512
NV = COLS // L
NC = 2
NS = 16
NW = NC * NS
RPW = ROWS // NW
CHUNK = 16
NCH = RPW // CHUNK


def _sort16(x, up):
    if up:
        return lax.sort(x, dimension=0)
    return plsc.sort_key_val(x, x, descending=True)[0]


def _bitonic_sort_vregs(xs, asc):
    nv = len(xs)

    def up_of(r, k):
        u = ((r * L) & k) == 0
        return u if asc else (not u)

    xs = [_sort16(x, up_of(r, L)) for r, x in enumerate(xs)]
    k = 2 * L
    while k <= nv * L:
        j = k // 2
        while j >= L:
            dv = j // L
            new = list(xs)
            for r in range(nv):
                r2 = r ^ dv
                if r2 > r:
                    up = up_of(r, k)
                    lo = jnp.minimum(xs[r], xs[r2])
                    hi = jnp.maximum(xs[r], xs[r2])
                    new[r], new[r2] = (lo, hi) if up else (hi, lo)
            xs = new
            j //= 2
        xs = [_sort16(x, up_of(r, k)) for r, x in enumerate(xs)]
        k *= 2
    return xs


def _merge_half(xs):
    nv = len(xs)
    j = nv * L // 2
    while j >= L:
        dv = j // L
        new = list(xs)
        for r in range(nv):
            r2 = r ^ dv
            if r2 > r:
                new[r] = jnp.minimum(xs[r], xs[r2])
                new[r2] = jnp.maximum(xs[r], xs[r2])
        xs = new
        j //= 2
    return [lax.sort(x, dimension=0) for x in xs]


def _losn_sc_body(
    data_hbm, wrev_hbm, bias_hbm, out_hbm, rows_v, w_v, bias_v, out_v, out_sc, dsem
):
    wid = lax.axis_index("s") * NC + lax.axis_index("c")
    base = wid * RPW
    pltpu.sync_copy(wrev_hbm, w_v)
    pltpu.sync_copy(bias_hbm, bias_v)
    bvec = bias_v[...]
    lanes = lax.broadcasted_iota(jnp.int32, (L,), 0)

    def _start(ch, slot):
        pltpu.async_copy(
            data_hbm.at[pl.ds(base + ch * CHUNK, CHUNK), :],
            rows_v.at[slot],
            dsem.at[slot],
        )

    _start(0, 0)

    def chunk_body(ch, carry):
        slot = ch % 2

        @pl.when(ch + 1 < NCH)
        def _():
            _start(ch + 1, (ch + 1) % 2)

        pltpu.make_async_copy(
            data_hbm.at[pl.ds(base, CHUNK), :],
            rows_v.at[slot],
            dsem.at[slot],
        ).wait()

        def one_row(t):
            xs = [rows_v[slot, t, pl.ds(v * L, L)] for v in range(NV)]
            xs = _bitonic_sort_vregs(xs, True)
            acc = xs[0] * w_v[pl.ds(0, L)]
            for v in range(1, NV):
                acc = acc + xs[v] * w_v[pl.ds(v * L, L)]
            return jnp.sum(acc)

        @plsc.parallel_loop(0, CHUNK, unroll=1)
        def _(t):
            total = one_row(t)
            out_sc[pl.ds((ch * CHUNK + t) * L, L)] = jnp.full((L,), total)

        return carry

    lax.fori_loop(0, NCH, chunk_body, jnp.zeros((), jnp.int32))

    def comp_body(g, carry):
        idx = g * (L * L) + lanes * L
        vals = plsc.load_gather(out_sc, [idx])
        out_v[pl.ds(g * L, L)] = vals + bvec
        return carry

    lax.fori_loop(0, RPW // L, comp_body, jnp.zeros((), jnp.int32))
    pltpu.sync_copy(out_v, out_hbm.at[pl.ds(base, RPW)])


@functools.lru_cache(maxsize=1)
def _build():
    mesh = plsc.VectorSubcoreMesh(
        core_axis_name="c", subcore_axis_name="s", num_cores=NC, num_subcores=NS
    )
    return pl.kernel(
        _losn_sc_body,
        out_type=jax.ShapeDtypeStruct((ROWS,), jnp.float32),
        mesh=mesh,
        compiler_params=pltpu.CompilerParams(needs_layout_passes=False),
        scratch_types=[
            pltpu.VMEM((2, CHUNK, COLS), jnp.float32),
            pltpu.VMEM((COLS,), jnp.float32),
            pltpu.VMEM((L,), jnp.float32),
            pltpu.VMEM((RPW,), jnp.float32),
            pltpu.VMEM((RPW * L,), jnp.float32),
            pltpu.SemaphoreType.DMA((2,)),
        ],
    )


def kernel(data, weights, bias):
    wrev = jnp.flip(weights.reshape(COLS))
    bias16 = jnp.broadcast_to(bias.reshape(()), (L,))
    out = _build()(data, wrev, bias16)
    return out.reshape(ROWS, 1)
